# baseline (speedup 1.0000x reference)
import functools
import math

import jax
import jax.numpy as jnp
from jax import lax
from jax.experimental import pallas as pl
from jax.experimental.pallas import tpu as pltpu

_GRP = 16


def _diag_mask(K, C2):
    rows = lax.broadcasted_iota(jnp.int32, (_GRP * K, _GRP * C2), 0) // K
    cols = lax.broadcasted_iota(jnp.int32, (_GRP * K, _GRP * C2), 1) // C2
    return (rows == cols).astype(jnp.bfloat16)


def _blockdiag(wn, mask):
    tiled = jnp.concatenate([wn] * _GRP, axis=1)
    return tiled * mask


def _body(x_ref, ip_ref, w_ref, rp_ref, o_ref, cur_ref, m_ref, wg_ref,
          wgi_ref, *, V, K, C, Bt, L, NG):
    C2 = K * K

    @pl.when(pl.program_id(0) == 0)
    def _prep():
        maskw = _diag_mask(K, C2)

        def sum_grp(gi, _):
            w = w_ref[pl.ds(gi * _GRP, _GRP)]
            wm = jnp.max(w, axis=-1, keepdims=True)
            wl = jnp.log(jnp.sum(jnp.exp(w - wm), axis=-1, keepdims=True)) + wm
            wn = jnp.exp(w - wl).reshape(_GRP * K, C2).astype(jnp.bfloat16)
            wg_ref[gi] = _blockdiag(wn, maskw)
            return 0
        jax.lax.fori_loop(0, NG, sum_grp, 0, unroll=2)

        maskl = _diag_mask(K, C)

        def leaf_grp(gi, _):
            ip = ip_ref[pl.ds(gi * _GRP, _GRP)]
            m = jnp.max(ip, axis=-1, keepdims=True)
            lse = jnp.log(jnp.sum(jnp.exp(ip - m), axis=-1, keepdims=True)) + m
            ipn = jnp.exp(ip - lse).reshape(_GRP * K, C).astype(jnp.bfloat16)
            wgi_ref[gi] = _blockdiag(ipn, maskl)
            return 0
        jax.lax.fori_loop(0, V // _GRP, leaf_grp, 0, unroll=2)

    cc = lax.broadcasted_iota(jnp.int16, (_GRP, C, Bt), 1)

    def gather_chunk(gi, _):
        X = x_ref[pl.ds(gi * _GRP, _GRP), :]
        oh = (X[:, None, :] == cc).astype(jnp.bfloat16)
        ohb = oh.reshape(_GRP * C, Bt)
        Wi = wgi_ref[gi]
        o = lax.dot_general(Wi, ohb, (((1,), (0,)), ((), ())),
                            preferred_element_type=jnp.float32)
        cur_ref[pl.ds(gi * _GRP, _GRP)] = o.reshape(_GRP, K, Bt).astype(jnp.bfloat16)
        return 0
    jax.lax.fori_loop(0, V // _GRP, gather_chunk, 0, unroll=4)

    R = V
    goff = 0
    first = True
    for _ in range(L):
        Rn = R // 2
        if Rn < _GRP:
            break

        def layer_chunk(ci, _, goff=goff, first=first):
            r0 = ci * _GRP
            p = cur_ref[pl.ds(2 * r0, 2 * _GRP)].reshape(_GRP, 2, K, Bt)
            left = p[:, 0]
            right = p[:, 1]
            E = jnp.concatenate(
                [left[:, i, :][:, None, :] * right for i in range(K)], axis=1)
            Eb = E.reshape(_GRP * K * K, Bt)
            Wb = wg_ref[goff + ci]
            o = lax.dot_general(Wb, Eb, (((1,), (0,)), ((), ())),
                                preferred_element_type=jnp.float32)
            o = o.reshape(_GRP, K, Bt)
            mo = jnp.max(o, axis=1, keepdims=True)
            cur_ref[pl.ds(r0, _GRP)] = (o * (1.0 / mo)).astype(jnp.bfloat16)
            lm = jnp.log(mo)[:, 0, :]
            if first:
                m_ref[pl.ds(r0, _GRP), :] = lm
            else:
                mp = m_ref[pl.ds(2 * r0, 2 * _GRP), :].reshape(_GRP, 2, Bt)
                m_ref[pl.ds(r0, _GRP), :] = mp[:, 0] + mp[:, 1] + lm
            return 0

        jax.lax.fori_loop(0, Rn // _GRP, layer_chunk, 0,
                          unroll=min(4, Rn // _GRP))
        goff += Rn // _GRP
        R = Rn
        first = False

    off = V - R
    while R > 1:
        Rn = R // 2
        p = cur_ref[pl.ds(0, 2 * Rn)].reshape(Rn, 2, K, Bt).astype(jnp.float32)
        left = p[:, 0]
        right = p[:, 1]
        w = w_ref[pl.ds(off, Rn)]
        wm = jnp.max(w, axis=-1, keepdims=True)
        wl = jnp.log(jnp.sum(jnp.exp(w - wm), axis=-1, keepdims=True)) + wm
        Wn = jnp.exp(w - wl)
        acc = None
        for i in range(K):
            t = None
            for j in range(K):
                term = Wn[:, :, i * K + j][:, :, None] * right[:, j, :][:, None, :]
                t = term if t is None else t + term
            contrib = left[:, i, :][:, None, :] * t
            acc = contrib if acc is None else acc + contrib
        mo = jnp.max(acc, axis=1, keepdims=True)
        cur_ref[pl.ds(0, Rn)] = (acc * (1.0 / mo)).astype(jnp.bfloat16)
        mp = m_ref[pl.ds(0, 2 * Rn), :].reshape(Rn, 2, Bt)
        m_ref[pl.ds(0, Rn), :] = mp[:, 0] + mp[:, 1] + jnp.log(mo)[:, 0, :]
        off += Rn
        R = Rn

    rp = rp_ref[...]
    rm = jnp.max(rp, axis=0, keepdims=True)
    rl = jnp.log(jnp.sum(jnp.exp(rp - rm), axis=0, keepdims=True)) + rm
    wr = jnp.exp(rp - rl)
    s = jnp.sum(wr * cur_ref[0].astype(jnp.float32), axis=0, keepdims=True)
    lls = jnp.log(s) + m_ref[pl.ds(0, 1), :]
    o_ref[...] = lls[None]


def kernel(inputs, input_params, sum_params, root_params):
    B, V = inputs.shape
    _, K, C = input_params.shape
    C2 = K * K
    L = int(math.log2(V))
    Bt = 256
    G = B // Bt
    NG = sum(
        (V >> (l + 1)) // _GRP for l in range(L) if (V >> (l + 1)) >= _GRP)
    NGI = V // _GRP

    xT = inputs.T.astype(jnp.int16)
    rpb = jnp.broadcast_to(root_params[:, None], (K, B))

    body = functools.partial(_body, V=V, K=K, C=C, Bt=Bt, L=L, NG=NG)
    out = pl.pallas_call(
        body,
        grid=(G,),
        in_specs=[
            pl.BlockSpec((V, Bt), lambda g: (0, g)),
            pl.BlockSpec((V, K, C), lambda g: (0, 0, 0)),
            pl.BlockSpec((V - 1, K, K * K), lambda g: (0, 0, 0)),
            pl.BlockSpec((K, Bt), lambda g: (0, g)),
        ],
        out_specs=pl.BlockSpec((1, 1, Bt), lambda g: (g, 0, 0)),
        out_shape=jax.ShapeDtypeStruct((G, 1, Bt), jnp.float32),
        scratch_shapes=[
            pltpu.VMEM((V, K, Bt), jnp.bfloat16),
            pltpu.VMEM((V, Bt), jnp.float32),
            pltpu.VMEM((NG, _GRP * K, _GRP * C2), jnp.bfloat16),
            pltpu.VMEM((NGI, _GRP * K, _GRP * C), jnp.bfloat16),
        ],
        compiler_params=pltpu.CompilerParams(
            dimension_semantics=("arbitrary",),
        ),
    )(xT, input_params, sum_params, rpb)
    return out.reshape(B, 1)


# speedup vs baseline: 1.0162x; 1.0162x over previous
import functools
import math

import jax
import jax.numpy as jnp
from jax import lax
from jax.experimental import pallas as pl
from jax.experimental.pallas import tpu as pltpu

_GRP = 16


def _diag_mask(K, C2):
    rows = lax.broadcasted_iota(jnp.int32, (_GRP * K, _GRP * C2), 0) // K
    cols = lax.broadcasted_iota(jnp.int32, (_GRP * K, _GRP * C2), 1) // C2
    return (rows == cols).astype(jnp.bfloat16)


def _blockdiag(wn, mask):
    tiled = jnp.concatenate([wn] * _GRP, axis=1)
    return tiled * mask


def _body(x_ref, ip_ref, w_ref, rp_ref, o_ref, cur_ref, m_ref, wg_ref,
          wgi_ref, *, V, K, C, Bt, L, NG):
    C2 = K * K

    @pl.when(pl.program_id(0) == 0)
    def _prep():
        maskw = _diag_mask(K, C2)

        def sum_grp(gi, _):
            w = w_ref[pl.ds(gi * _GRP, _GRP)]
            wm = jnp.max(w, axis=-1, keepdims=True)
            wl = jnp.log(jnp.sum(jnp.exp(w - wm), axis=-1, keepdims=True)) + wm
            wn = jnp.exp(w - wl).reshape(_GRP * K, C2).astype(jnp.bfloat16)
            wg_ref[gi] = _blockdiag(wn, maskw)
            return 0
        jax.lax.fori_loop(0, NG, sum_grp, 0, unroll=2)

        maskl = _diag_mask(K, C)

        def leaf_grp(gi, _):
            ip = ip_ref[pl.ds(gi * _GRP, _GRP)]
            m = jnp.max(ip, axis=-1, keepdims=True)
            lse = jnp.log(jnp.sum(jnp.exp(ip - m), axis=-1, keepdims=True)) + m
            ipn = jnp.exp(ip - lse).reshape(_GRP * K, C).astype(jnp.bfloat16)
            wgi_ref[gi] = _blockdiag(ipn, maskl)
            return 0
        jax.lax.fori_loop(0, V // _GRP, leaf_grp, 0, unroll=2)

    cc = lax.broadcasted_iota(jnp.int16, (_GRP, C, Bt), 1)

    def gather_chunk(gi, _):
        X = x_ref[pl.ds(gi * _GRP, _GRP), :]
        oh = (X[:, None, :] == cc).astype(jnp.bfloat16)
        ohb = oh.reshape(_GRP * C, Bt)
        Wi = wgi_ref[gi]
        o = lax.dot_general(Wi, ohb, (((1,), (0,)), ((), ())),
                            preferred_element_type=jnp.float32)
        cur_ref[pl.ds(gi * _GRP, _GRP)] = o.reshape(_GRP, K, Bt)
        return 0
    jax.lax.fori_loop(0, V // _GRP, gather_chunk, 0, unroll=4)

    R = V
    goff = 0
    first = True
    for _ in range(L):
        Rn = R // 2
        if Rn < _GRP:
            break

        def layer_chunk(ci, _, goff=goff, first=first):
            r0 = ci * _GRP
            p = cur_ref[pl.ds(2 * r0, 2 * _GRP)].reshape(_GRP, 2, K, Bt)
            left = p[:, 0]
            right = p[:, 1]
            E = jnp.concatenate(
                [left[:, i, :][:, None, :] * right for i in range(K)], axis=1)
            Eb = E.reshape(_GRP * K * K, Bt).astype(jnp.bfloat16)
            Wb = wg_ref[goff + ci]
            o = lax.dot_general(Wb, Eb, (((1,), (0,)), ((), ())),
                                preferred_element_type=jnp.float32)
            o = o.reshape(_GRP, K, Bt)
            mo = jnp.max(o, axis=1, keepdims=True)
            cur_ref[pl.ds(r0, _GRP)] = o * (1.0 / mo)
            lm = jnp.log(mo)[:, 0, :]
            if first:
                m_ref[pl.ds(r0, _GRP), :] = lm
            else:
                mp = m_ref[pl.ds(2 * r0, 2 * _GRP), :].reshape(_GRP, 2, Bt)
                m_ref[pl.ds(r0, _GRP), :] = mp[:, 0] + mp[:, 1] + lm
            return 0

        jax.lax.fori_loop(0, Rn // _GRP, layer_chunk, 0,
                          unroll=min(4, Rn // _GRP))
        goff += Rn // _GRP
        R = Rn
        first = False

    off = V - R
    while R > 1:
        Rn = R // 2
        p = cur_ref[pl.ds(0, 2 * Rn)].reshape(Rn, 2, K, Bt)
        left = p[:, 0]
        right = p[:, 1]
        w = w_ref[pl.ds(off, Rn)]
        wm = jnp.max(w, axis=-1, keepdims=True)
        wl = jnp.log(jnp.sum(jnp.exp(w - wm), axis=-1, keepdims=True)) + wm
        Wn = jnp.exp(w - wl)
        acc = None
        for i in range(K):
            t = None
            for j in range(K):
                term = Wn[:, :, i * K + j][:, :, None] * right[:, j, :][:, None, :]
                t = term if t is None else t + term
            contrib = left[:, i, :][:, None, :] * t
            acc = contrib if acc is None else acc + contrib
        mo = jnp.max(acc, axis=1, keepdims=True)
        cur_ref[pl.ds(0, Rn)] = acc * (1.0 / mo)
        mp = m_ref[pl.ds(0, 2 * Rn), :].reshape(Rn, 2, Bt)
        m_ref[pl.ds(0, Rn), :] = mp[:, 0] + mp[:, 1] + jnp.log(mo)[:, 0, :]
        off += Rn
        R = Rn

    rp = rp_ref[...]
    rm = jnp.max(rp, axis=0, keepdims=True)
    rl = jnp.log(jnp.sum(jnp.exp(rp - rm), axis=0, keepdims=True)) + rm
    wr = jnp.exp(rp - rl)
    s = jnp.sum(wr * cur_ref[0], axis=0, keepdims=True)
    lls = jnp.log(s) + m_ref[pl.ds(0, 1), :]
    o_ref[...] = lls[None]


def kernel(inputs, input_params, sum_params, root_params):
    B, V = inputs.shape
    _, K, C = input_params.shape
    C2 = K * K
    L = int(math.log2(V))
    Bt = 256
    G = B // Bt
    NG = sum(
        (V >> (l + 1)) // _GRP for l in range(L) if (V >> (l + 1)) >= _GRP)
    NGI = V // _GRP

    xT = inputs.T.astype(jnp.int16)
    rpb = jnp.broadcast_to(root_params[:, None], (K, B))

    body = functools.partial(_body, V=V, K=K, C=C, Bt=Bt, L=L, NG=NG)
    out = pl.pallas_call(
        body,
        grid=(G,),
        in_specs=[
            pl.BlockSpec((V, Bt), lambda g: (0, g)),
            pl.BlockSpec((V, K, C), lambda g: (0, 0, 0)),
            pl.BlockSpec((V - 1, K, K * K), lambda g: (0, 0, 0)),
            pl.BlockSpec((K, Bt), lambda g: (0, g)),
        ],
        out_specs=pl.BlockSpec((1, 1, Bt), lambda g: (g, 0, 0)),
        out_shape=jax.ShapeDtypeStruct((G, 1, Bt), jnp.float32),
        scratch_shapes=[
            pltpu.VMEM((V, K, Bt), jnp.float32),
            pltpu.VMEM((V, Bt), jnp.float32),
            pltpu.VMEM((NG, _GRP * K, _GRP * C2), jnp.bfloat16),
            pltpu.VMEM((NGI, _GRP * K, _GRP * C), jnp.bfloat16),
        ],
        compiler_params=pltpu.CompilerParams(
            dimension_semantics=("arbitrary",),
        ),
    )(xT, input_params, sum_params, rpb)
    return out.reshape(B, 1)


# speedup vs baseline: 1.2684x; 1.2481x over previous
import functools
import math

import jax
import jax.numpy as jnp
from jax import lax
from jax.experimental import pallas as pl
from jax.experimental.pallas import tpu as pltpu

_GRP = 16


def _diag_mask(K, C2):
    rows = lax.broadcasted_iota(jnp.int32, (_GRP * K, _GRP * C2), 0) // K
    cols = lax.broadcasted_iota(jnp.int32, (_GRP * K, _GRP * C2), 1) // C2
    return (rows == cols).astype(jnp.bfloat16)


def _blockdiag(wn, mask):
    tiled = jnp.concatenate([wn] * _GRP, axis=1)
    return tiled * mask


def _body(x_ref, ip_ref, w_ref, rp_ref, o_ref, cur_ref, m_ref, wg_ref,
          wgi_ref, *, V, K, C, Bt, L, NG):
    C2 = K * K

    @pl.when(pl.program_id(0) == 0)
    def _prep():
        maskw = _diag_mask(K, C2)

        def sum_grp(gi, _):
            w = w_ref[pl.ds(gi * _GRP, _GRP)]
            wm = jnp.max(w, axis=-1, keepdims=True)
            wl = jnp.log(jnp.sum(jnp.exp(w - wm), axis=-1, keepdims=True)) + wm
            wn = jnp.exp(w - wl).reshape(_GRP * K, C2).astype(jnp.bfloat16)
            wg_ref[gi] = _blockdiag(wn, maskw)
            return 0
        jax.lax.fori_loop(0, NG, sum_grp, 0, unroll=2)

        maskl = _diag_mask(K, C)

        def leaf_grp(gi, _):
            ip = ip_ref[pl.ds(gi * _GRP, _GRP)]
            m = jnp.max(ip, axis=-1, keepdims=True)
            lse = jnp.log(jnp.sum(jnp.exp(ip - m), axis=-1, keepdims=True)) + m
            ipn = jnp.exp(ip - lse).reshape(_GRP * K, C).astype(jnp.bfloat16)
            wgi_ref[gi] = _blockdiag(ipn, maskl)
            return 0
        jax.lax.fori_loop(0, V // _GRP, leaf_grp, 0, unroll=2)

    cc = lax.broadcasted_iota(jnp.int32, (_GRP, C, Bt), 1)

    def gather_chunk(gi, _):
        X = x_ref[pl.ds(gi * _GRP, _GRP), :]
        oh = (X[:, None, :] == cc).astype(jnp.bfloat16)
        ohb = oh.reshape(_GRP * C, Bt)
        Wi = wgi_ref[gi]
        o = lax.dot_general(Wi, ohb, (((1,), (0,)), ((), ())),
                            preferred_element_type=jnp.float32)
        cur_ref[pl.ds(gi * _GRP, _GRP)] = o.reshape(_GRP, K, Bt)
        return 0
    jax.lax.fori_loop(0, V // _GRP, gather_chunk, 0, unroll=4)

    R = V
    goff = 0
    first = True
    for _ in range(L):
        Rn = R // 2
        if Rn < _GRP:
            break

        def layer_chunk(ci, _, goff=goff, first=first):
            r0 = ci * _GRP
            p = cur_ref[pl.ds(2 * r0, 2 * _GRP)].reshape(_GRP, 2, K, Bt)
            left = p[:, 0]
            right = p[:, 1]
            E = jnp.concatenate(
                [left[:, i, :][:, None, :] * right for i in range(K)], axis=1)
            Eb = E.reshape(_GRP * K * K, Bt).astype(jnp.bfloat16)
            Wb = wg_ref[goff + ci]
            o = lax.dot_general(Wb, Eb, (((1,), (0,)), ((), ())),
                                preferred_element_type=jnp.float32)
            o = o.reshape(_GRP, K, Bt)
            mo = jnp.max(o, axis=1, keepdims=True)
            cur_ref[pl.ds(r0, _GRP)] = o * (1.0 / mo)
            lm = jnp.log(mo)[:, 0, :]
            if first:
                m_ref[pl.ds(r0, _GRP), :] = lm
            else:
                mp = m_ref[pl.ds(2 * r0, 2 * _GRP), :].reshape(_GRP, 2, Bt)
                m_ref[pl.ds(r0, _GRP), :] = mp[:, 0] + mp[:, 1] + lm
            return 0

        jax.lax.fori_loop(0, Rn // _GRP, layer_chunk, 0,
                          unroll=min(4, Rn // _GRP))
        goff += Rn // _GRP
        R = Rn
        first = False

    off = V - R
    while R > 1:
        Rn = R // 2
        p = cur_ref[pl.ds(0, 2 * Rn)].reshape(Rn, 2, K, Bt)
        left = p[:, 0]
        right = p[:, 1]
        w = w_ref[pl.ds(off, Rn)]
        wm = jnp.max(w, axis=-1, keepdims=True)
        wl = jnp.log(jnp.sum(jnp.exp(w - wm), axis=-1, keepdims=True)) + wm
        Wn = jnp.exp(w - wl)
        acc = None
        for i in range(K):
            t = None
            for j in range(K):
                term = Wn[:, :, i * K + j][:, :, None] * right[:, j, :][:, None, :]
                t = term if t is None else t + term
            contrib = left[:, i, :][:, None, :] * t
            acc = contrib if acc is None else acc + contrib
        mo = jnp.max(acc, axis=1, keepdims=True)
        cur_ref[pl.ds(0, Rn)] = acc * (1.0 / mo)
        mp = m_ref[pl.ds(0, 2 * Rn), :].reshape(Rn, 2, Bt)
        m_ref[pl.ds(0, Rn), :] = mp[:, 0] + mp[:, 1] + jnp.log(mo)[:, 0, :]
        off += Rn
        R = Rn

    rp = rp_ref[...]
    rm = jnp.max(rp, axis=0, keepdims=True)
    rl = jnp.log(jnp.sum(jnp.exp(rp - rm), axis=0, keepdims=True)) + rm
    wr = jnp.exp(rp - rl)
    s = jnp.sum(wr * cur_ref[0], axis=0, keepdims=True)
    lls = jnp.log(s) + m_ref[pl.ds(0, 1), :]
    o_ref[...] = lls[None]


def kernel(inputs, input_params, sum_params, root_params):
    B, V = inputs.shape
    _, K, C = input_params.shape
    C2 = K * K
    L = int(math.log2(V))
    Bt = 256
    G = B // Bt
    NG = sum(
        (V >> (l + 1)) // _GRP for l in range(L) if (V >> (l + 1)) >= _GRP)
    NGI = V // _GRP

    xT = inputs.T
    rpb = jnp.broadcast_to(root_params[:, None], (K, B))

    body = functools.partial(_body, V=V, K=K, C=C, Bt=Bt, L=L, NG=NG)
    out = pl.pallas_call(
        body,
        grid=(G,),
        in_specs=[
            pl.BlockSpec((V, Bt), lambda g: (0, g)),
            pl.BlockSpec((V, K, C), lambda g: (0, 0, 0)),
            pl.BlockSpec((V - 1, K, K * K), lambda g: (0, 0, 0)),
            pl.BlockSpec((K, Bt), lambda g: (0, g)),
        ],
        out_specs=pl.BlockSpec((1, 1, Bt), lambda g: (g, 0, 0)),
        out_shape=jax.ShapeDtypeStruct((G, 1, Bt), jnp.float32),
        scratch_shapes=[
            pltpu.VMEM((V, K, Bt), jnp.float32),
            pltpu.VMEM((V, Bt), jnp.float32),
            pltpu.VMEM((NG, _GRP * K, _GRP * C2), jnp.bfloat16),
            pltpu.VMEM((NGI, _GRP * K, _GRP * C), jnp.bfloat16),
        ],
        compiler_params=pltpu.CompilerParams(
            dimension_semantics=("arbitrary",),
        ),
    )(xT, input_params, sum_params, rpb)
    return out.reshape(B, 1)
